# Initial kernel scaffold; baseline (speedup 1.0000x reference)
import jax
import jax.numpy as jnp
from jax.experimental import pallas as pl


def kernel(node_ids, edge_index, emb, W1, att_src1, att_dst1, b1, W2, att_src2, att_dst2, b2, fc_W, fc_b, mu_W, mu_b, var_W, var_b, pi_W, pi_b):
    raise NotImplementedError("write your pallas kernel here")


# speedup vs baseline: 2.4248x; 2.4248x over previous
import jax
import jax.numpy as jnp
from jax.experimental import pallas as pl


_NB = 1000


def _elu(x):
    return jnp.where(x > 0, x, jnp.exp(jnp.minimum(x, 0.0)) - 1.0)


def _dense1_body(x_ref, w1_ref, asrc_ref, adst_ref, sel_ref,
                 h_ref, as_ref, ad_ref):
    h = jnp.dot(x_ref[...], w1_ref[...], preferred_element_type=jnp.float32)
    h_ref[...] = h
    sel = sel_ref[...]
    as_ref[...] = jnp.dot(h * asrc_ref[...], sel,
                          preferred_element_type=jnp.float32)
    ad_ref[...] = jnp.dot(h * adst_ref[...], sel,
                          preferred_element_type=jnp.float32)


def _dense2_body(agg_ref, b1_ref, w2_ref, asrc_ref, adst_ref, ones_ref,
                 h_ref, as_ref, ad_ref):
    x = _elu(agg_ref[...] + b1_ref[...])
    h = jnp.dot(x, w2_ref[...], preferred_element_type=jnp.float32)
    h_ref[...] = h
    ones = ones_ref[...]
    as_ref[...] = jnp.dot(h * asrc_ref[...], ones,
                          preferred_element_type=jnp.float32)
    ad_ref[...] = jnp.dot(h * adst_ref[...], ones,
                          preferred_element_type=jnp.float32)


def _heads_body(agg_ref, b2_ref, fcw_ref, fcb_ref, muw_ref, mub_ref,
                varw_ref, varb_ref, piw_ref, pib_ref,
                mu_ref, var_ref, pi_ref):
    x = _elu(agg_ref[...] + b2_ref[...])
    f = _elu(jnp.dot(x, fcw_ref[...], preferred_element_type=jnp.float32)
             + fcb_ref[...])
    mu_ref[...] = jnp.dot(f, muw_ref[...],
                          preferred_element_type=jnp.float32) + mub_ref[...]
    v = jnp.dot(f, varw_ref[...],
                preferred_element_type=jnp.float32) + varb_ref[...]
    var_ref[...] = jnp.where(v > 0, v, 0.0) + jnp.log(1.0 + jnp.exp(-jnp.abs(v)))
    p = jnp.dot(f, piw_ref[...],
                preferred_element_type=jnp.float32) + pib_ref[...]
    p = p - jnp.max(p, axis=-1, keepdims=True)
    e = jnp.exp(p)
    pi_ref[...] = e / jnp.sum(e, axis=-1, keepdims=True)


def _full(shape):
    return pl.BlockSpec(shape, lambda i: (0, 0))


def _rows(cols):
    return pl.BlockSpec((_NB, cols), lambda i: (i, 0))


def kernel(node_ids, edge_index, emb, W1, att_src1, att_dst1, b1,
           W2, att_src2, att_dst2, b2, fc_W, fc_b,
           mu_W, mu_b, var_W, var_b, pi_W, pi_b):
    N = node_ids.shape[0]
    D = emb.shape[1]
    heads, H = att_src1.shape[1], att_src1.shape[2]
    F1 = heads * H
    grid = (N // _NB,)

    x = emb[node_ids]

    sel = (jnp.arange(F1)[:, None] // H
           == jnp.arange(heads)[None, :]).astype(jnp.float32)

    h1, as1, ad1 = pl.pallas_call(
        _dense1_body,
        grid=grid,
        in_specs=[_rows(D), _full((D, F1)), _full((1, F1)), _full((1, F1)),
                  _full((F1, heads))],
        out_specs=[_rows(F1), _rows(heads), _rows(heads)],
        out_shape=[jax.ShapeDtypeStruct((N, F1), jnp.float32),
                   jax.ShapeDtypeStruct((N, heads), jnp.float32),
                   jax.ShapeDtypeStruct((N, heads), jnp.float32)],
    )(x, W1, att_src1.reshape(1, F1), att_dst1.reshape(1, F1), sel)

    loops = jnp.arange(N, dtype=edge_index.dtype)
    ei = jnp.concatenate([edge_index, jnp.stack([loops, loops])], axis=1)
    src, dst = ei[0], ei[1]

    def edge_softmax(a_src, a_dst, h, nheads, hdim):
        alpha = a_src[src] + a_dst[dst]
        alpha = jnp.where(alpha > 0, alpha, 0.2 * alpha)
        amax = jax.ops.segment_max(alpha, dst, num_segments=N)
        alpha = jnp.exp(alpha - amax[dst])
        denom = jax.ops.segment_sum(alpha, dst, num_segments=N)
        alpha = alpha / (denom[dst] + 1e-16)
        msg = (h[src].reshape(-1, nheads, hdim) * alpha[:, :, None])
        return jax.ops.segment_sum(msg.reshape(-1, nheads * hdim), dst,
                                   num_segments=N)

    agg1 = edge_softmax(as1, ad1, h1, heads, H)

    h2, as2, ad2 = pl.pallas_call(
        _dense2_body,
        grid=grid,
        in_specs=[_rows(F1), _full((1, F1)), _full((F1, H)), _full((1, H)),
                  _full((1, H)), _full((H, 1))],
        out_specs=[_rows(H), _rows(1), _rows(1)],
        out_shape=[jax.ShapeDtypeStruct((N, H), jnp.float32),
                   jax.ShapeDtypeStruct((N, 1), jnp.float32),
                   jax.ShapeDtypeStruct((N, 1), jnp.float32)],
    )(agg1, b1.reshape(1, F1), W2, att_src2.reshape(1, H),
      att_dst2.reshape(1, H), jnp.ones((H, 1), jnp.float32))

    agg2 = edge_softmax(as2, ad2, h2, 1, H)

    M = mu_W.shape[1]
    mu, var, pi = pl.pallas_call(
        _heads_body,
        grid=grid,
        in_specs=[_rows(H), _full((1, H)), _full((H, H)), _full((1, H)),
                  _full((H, M)), _full((1, M)), _full((H, M)), _full((1, M)),
                  _full((H, M)), _full((1, M))],
        out_specs=[_rows(M), _rows(M), _rows(M)],
        out_shape=[jax.ShapeDtypeStruct((N, M), jnp.float32),
                   jax.ShapeDtypeStruct((N, M), jnp.float32),
                   jax.ShapeDtypeStruct((N, M), jnp.float32)],
    )(agg2, b2.reshape(1, H), fc_W, fc_b.reshape(1, H),
      mu_W, mu_b.reshape(1, M), var_W, var_b.reshape(1, M),
      pi_W, pi_b.reshape(1, M))

    return (mu, var, pi)


# speedup vs baseline: 2.6660x; 1.0995x over previous
import jax
import jax.numpy as jnp
from jax.experimental import pallas as pl


_NB = 1000


def _elu(x):
    return jnp.where(x > 0, x, jnp.exp(jnp.minimum(x, 0.0)) - 1.0)


def _dense1_body(x_ref, w1_ref, asrc_ref, adst_ref, sel_ref,
                 h_ref, as_ref, ad_ref):
    h = jnp.dot(x_ref[...], w1_ref[...], preferred_element_type=jnp.float32)
    h_ref[...] = h
    sel = sel_ref[...]
    as_ref[...] = jnp.dot(h * asrc_ref[...], sel,
                          preferred_element_type=jnp.float32)
    ad_ref[...] = jnp.dot(h * adst_ref[...], sel,
                          preferred_element_type=jnp.float32)


def _dense2_body(agg_ref, b1_ref, w2_ref, asrc_ref, adst_ref, ones_ref,
                 h_ref, as_ref, ad_ref):
    x = _elu(agg_ref[...] + b1_ref[...])
    h = jnp.dot(x, w2_ref[...], preferred_element_type=jnp.float32)
    h_ref[...] = h
    ones = ones_ref[...]
    as_ref[...] = jnp.dot(h * asrc_ref[...], ones,
                          preferred_element_type=jnp.float32)
    ad_ref[...] = jnp.dot(h * adst_ref[...], ones,
                          preferred_element_type=jnp.float32)


def _heads_body(agg_ref, b2_ref, fcw_ref, fcb_ref, muw_ref, mub_ref,
                varw_ref, varb_ref, piw_ref, pib_ref,
                mu_ref, var_ref, pi_ref):
    x = _elu(agg_ref[...] + b2_ref[...])
    f = _elu(jnp.dot(x, fcw_ref[...], preferred_element_type=jnp.float32)
             + fcb_ref[...])
    mu_ref[...] = jnp.dot(f, muw_ref[...],
                          preferred_element_type=jnp.float32) + mub_ref[...]
    v = jnp.dot(f, varw_ref[...],
                preferred_element_type=jnp.float32) + varb_ref[...]
    var_ref[...] = jnp.where(v > 0, v, 0.0) + jnp.log(1.0 + jnp.exp(-jnp.abs(v)))
    p = jnp.dot(f, piw_ref[...],
                preferred_element_type=jnp.float32) + pib_ref[...]
    p = p - jnp.max(p, axis=-1, keepdims=True)
    e = jnp.exp(p)
    pi_ref[...] = e / jnp.sum(e, axis=-1, keepdims=True)


def _full(shape):
    return pl.BlockSpec(shape, lambda i: (0, 0))


def _rows(cols):
    return pl.BlockSpec((_NB, cols), lambda i: (i, 0))


def kernel(node_ids, edge_index, emb, W1, att_src1, att_dst1, b1,
           W2, att_src2, att_dst2, b2, fc_W, fc_b,
           mu_W, mu_b, var_W, var_b, pi_W, pi_b):
    N = node_ids.shape[0]
    D = emb.shape[1]
    heads, H = att_src1.shape[1], att_src1.shape[2]
    F1 = heads * H
    grid = (N // _NB,)

    x = emb[node_ids]

    sel = (jnp.arange(F1)[:, None] // H
           == jnp.arange(heads)[None, :]).astype(jnp.float32)

    h1, as1, ad1 = pl.pallas_call(
        _dense1_body,
        grid=grid,
        in_specs=[_rows(D), _full((D, F1)), _full((1, F1)), _full((1, F1)),
                  _full((F1, heads))],
        out_specs=[_rows(F1), _rows(heads), _rows(heads)],
        out_shape=[jax.ShapeDtypeStruct((N, F1), jnp.float32),
                   jax.ShapeDtypeStruct((N, heads), jnp.float32),
                   jax.ShapeDtypeStruct((N, heads), jnp.float32)],
    )(x, W1, att_src1.reshape(1, F1), att_dst1.reshape(1, F1), sel)

    loops = jnp.arange(N, dtype=edge_index.dtype)
    ei = jnp.concatenate([edge_index, jnp.stack([loops, loops])], axis=1)
    src, dst = ei[0], ei[1]

    def edge_softmax(a_src, a_dst, h, nheads, hdim):
        alpha = a_src[src] + a_dst[dst]
        alpha = jnp.where(alpha > 0, alpha, 0.2 * alpha)
        amax = jax.ops.segment_max(alpha, dst, num_segments=N)
        alpha = jnp.exp(alpha - amax[dst])
        denom = jax.ops.segment_sum(alpha, dst, num_segments=N)
        msg = (h[src].reshape(-1, nheads, hdim) * alpha[:, :, None])
        agg = jax.ops.segment_sum(msg.reshape(-1, nheads * hdim), dst,
                                  num_segments=N)
        scale = 1.0 / (denom + 1e-16)
        out = agg.reshape(N, nheads, hdim) * scale[:, :, None]
        return out.reshape(N, nheads * hdim)

    agg1 = edge_softmax(as1, ad1, h1, heads, H)

    h2, as2, ad2 = pl.pallas_call(
        _dense2_body,
        grid=grid,
        in_specs=[_rows(F1), _full((1, F1)), _full((F1, H)), _full((1, H)),
                  _full((1, H)), _full((H, 1))],
        out_specs=[_rows(H), _rows(1), _rows(1)],
        out_shape=[jax.ShapeDtypeStruct((N, H), jnp.float32),
                   jax.ShapeDtypeStruct((N, 1), jnp.float32),
                   jax.ShapeDtypeStruct((N, 1), jnp.float32)],
    )(agg1, b1.reshape(1, F1), W2, att_src2.reshape(1, H),
      att_dst2.reshape(1, H), jnp.ones((H, 1), jnp.float32))

    agg2 = edge_softmax(as2, ad2, h2, 1, H)

    M = mu_W.shape[1]
    mu, var, pi = pl.pallas_call(
        _heads_body,
        grid=grid,
        in_specs=[_rows(H), _full((1, H)), _full((H, H)), _full((1, H)),
                  _full((H, M)), _full((1, M)), _full((H, M)), _full((1, M)),
                  _full((H, M)), _full((1, M))],
        out_specs=[_rows(M), _rows(M), _rows(M)],
        out_shape=[jax.ShapeDtypeStruct((N, M), jnp.float32),
                   jax.ShapeDtypeStruct((N, M), jnp.float32),
                   jax.ShapeDtypeStruct((N, M), jnp.float32)],
    )(agg2, b2.reshape(1, H), fc_W, fc_b.reshape(1, H),
      mu_W, mu_b.reshape(1, M), var_W, var_b.reshape(1, M),
      pi_W, pi_b.reshape(1, M))

    return (mu, var, pi)


# speedup vs baseline: 2.6928x; 1.0101x over previous
import jax
import jax.numpy as jnp
from jax.experimental import pallas as pl


_NB = 1000


def _elu(x):
    return jnp.where(x > 0, x, jnp.exp(jnp.minimum(x, 0.0)) - 1.0)


def _dense1_body(x_ref, w1_ref, asrc_ref, adst_ref, sel_ref,
                 h_ref, as_ref, ad_ref):
    h = jnp.dot(x_ref[...], w1_ref[...], preferred_element_type=jnp.float32)
    h_ref[...] = h
    sel = sel_ref[...]
    as_ref[...] = jnp.dot(h * asrc_ref[...], sel,
                          preferred_element_type=jnp.float32)
    ad_ref[...] = jnp.dot(h * adst_ref[...], sel,
                          preferred_element_type=jnp.float32)


def _dense2_body(agg_ref, b1_ref, w2_ref, asrc_ref, adst_ref, ones_ref,
                 h_ref, as_ref, ad_ref):
    x = _elu(agg_ref[...] + b1_ref[...])
    h = jnp.dot(x, w2_ref[...], preferred_element_type=jnp.float32)
    h_ref[...] = h
    ones = ones_ref[...]
    as_ref[...] = jnp.dot(h * asrc_ref[...], ones,
                          preferred_element_type=jnp.float32)
    ad_ref[...] = jnp.dot(h * adst_ref[...], ones,
                          preferred_element_type=jnp.float32)


def _heads_body(agg_ref, b2_ref, fcw_ref, fcb_ref, muw_ref, mub_ref,
                varw_ref, varb_ref, piw_ref, pib_ref,
                mu_ref, var_ref, pi_ref):
    x = _elu(agg_ref[...] + b2_ref[...])
    f = _elu(jnp.dot(x, fcw_ref[...], preferred_element_type=jnp.float32)
             + fcb_ref[...])
    mu_ref[...] = jnp.dot(f, muw_ref[...],
                          preferred_element_type=jnp.float32) + mub_ref[...]
    v = jnp.dot(f, varw_ref[...],
                preferred_element_type=jnp.float32) + varb_ref[...]
    var_ref[...] = jnp.where(v > 0, v, 0.0) + jnp.log(1.0 + jnp.exp(-jnp.abs(v)))
    p = jnp.dot(f, piw_ref[...],
                preferred_element_type=jnp.float32) + pib_ref[...]
    p = p - jnp.max(p, axis=-1, keepdims=True)
    e = jnp.exp(p)
    pi_ref[...] = e / jnp.sum(e, axis=-1, keepdims=True)


def _full(shape):
    return pl.BlockSpec(shape, lambda i: (0, 0))


def _rows(cols):
    return pl.BlockSpec((_NB, cols), lambda i: (i, 0))


def kernel(node_ids, edge_index, emb, W1, att_src1, att_dst1, b1,
           W2, att_src2, att_dst2, b2, fc_W, fc_b,
           mu_W, mu_b, var_W, var_b, pi_W, pi_b):
    N = node_ids.shape[0]
    D = emb.shape[1]
    heads, H = att_src1.shape[1], att_src1.shape[2]
    F1 = heads * H
    grid = (N // _NB,)

    x = emb[node_ids]

    sel = (jnp.arange(F1)[:, None] // H
           == jnp.arange(heads)[None, :]).astype(jnp.float32)

    h1, as1, ad1 = pl.pallas_call(
        _dense1_body,
        grid=grid,
        in_specs=[_rows(D), _full((D, F1)), _full((1, F1)), _full((1, F1)),
                  _full((F1, heads))],
        out_specs=[_rows(F1), _rows(heads), _rows(heads)],
        out_shape=[jax.ShapeDtypeStruct((N, F1), jnp.float32),
                   jax.ShapeDtypeStruct((N, heads), jnp.float32),
                   jax.ShapeDtypeStruct((N, heads), jnp.float32)],
    )(x, W1, att_src1.reshape(1, F1), att_dst1.reshape(1, F1), sel)

    loops = jnp.arange(N, dtype=edge_index.dtype)
    ei = jnp.concatenate([edge_index, jnp.stack([loops, loops])], axis=1)
    src, dst = ei[0], ei[1]

    def edge_softmax(a_src, a_dst, h, nheads, hdim):
        alpha = a_src[src] + a_dst[dst]
        alpha = jnp.where(alpha > 0, alpha, 0.2 * alpha)
        amax = jax.ops.segment_max(alpha, dst, num_segments=N)
        alpha = jnp.exp(alpha - amax[dst])
        msg = (h[src].reshape(-1, nheads, hdim) * alpha[:, :, None])
        cat = jnp.concatenate([msg.reshape(-1, nheads * hdim), alpha], axis=1)
        aggc = jax.ops.segment_sum(cat, dst, num_segments=N)
        agg, denom = aggc[:, :nheads * hdim], aggc[:, nheads * hdim:]
        scale = 1.0 / (denom + 1e-16)
        out = agg.reshape(N, nheads, hdim) * scale[:, :, None]
        return out.reshape(N, nheads * hdim)

    agg1 = edge_softmax(as1, ad1, h1, heads, H)

    h2, as2, ad2 = pl.pallas_call(
        _dense2_body,
        grid=grid,
        in_specs=[_rows(F1), _full((1, F1)), _full((F1, H)), _full((1, H)),
                  _full((1, H)), _full((H, 1))],
        out_specs=[_rows(H), _rows(1), _rows(1)],
        out_shape=[jax.ShapeDtypeStruct((N, H), jnp.float32),
                   jax.ShapeDtypeStruct((N, 1), jnp.float32),
                   jax.ShapeDtypeStruct((N, 1), jnp.float32)],
    )(agg1, b1.reshape(1, F1), W2, att_src2.reshape(1, H),
      att_dst2.reshape(1, H), jnp.ones((H, 1), jnp.float32))

    agg2 = edge_softmax(as2, ad2, h2, 1, H)

    M = mu_W.shape[1]
    mu, var, pi = pl.pallas_call(
        _heads_body,
        grid=grid,
        in_specs=[_rows(H), _full((1, H)), _full((H, H)), _full((1, H)),
                  _full((H, M)), _full((1, M)), _full((H, M)), _full((1, M)),
                  _full((H, M)), _full((1, M))],
        out_specs=[_rows(M), _rows(M), _rows(M)],
        out_shape=[jax.ShapeDtypeStruct((N, M), jnp.float32),
                   jax.ShapeDtypeStruct((N, M), jnp.float32),
                   jax.ShapeDtypeStruct((N, M), jnp.float32)],
    )(agg2, b2.reshape(1, H), fc_W, fc_b.reshape(1, H),
      mu_W, mu_b.reshape(1, M), var_W, var_b.reshape(1, M),
      pi_W, pi_b.reshape(1, M))

    return (mu, var, pi)
